# baseline (speedup 1.0000x reference)
import functools

import jax
import jax.numpy as jnp
from jax import lax
from jax.experimental import pallas as pl
from jax.experimental.pallas import tpu as pltpu

NUM_NODES = 10000
NUM_EDGES = 320000
NUM_GRAPHS = 64
HID = 256


def _bn_cols(t, gamma, beta, eps=1e-5):
    mean = jnp.mean(t, axis=0, keepdims=True)
    var = jnp.mean((t - mean) ** 2, axis=0, keepdims=True)
    return gamma * (t - mean) * lax.rsqrt(var + eps) + beta


def _layer_body(h_ref, agg_ref, eps_ref, w1_ref, b1_ref, g1_ref, be1_ref,
                w2_ref, b2_ref, g2_ref, be2_ref, out_ref):
    eps = eps_ref[0, 0]
    z = (1.0 + eps) * h_ref[...] + agg_ref[...]
    t = jnp.dot(z, w1_ref[...], preferred_element_type=jnp.float32) + b1_ref[...]
    t = _bn_cols(t, g1_ref[...], be1_ref[...])
    t = jnp.maximum(t, 0.0)
    u = jnp.dot(t, w2_ref[...], preferred_element_type=jnp.float32) + b2_ref[...]
    u = _bn_cols(u, g2_ref[...], be2_ref[...])
    out_ref[...] = jnp.maximum(u, 0.0)


def _gin_layer(h, agg, p):
    n, _ = h.shape
    hid = p['W2'].shape[1]
    eps2d = p['eps'].reshape(1, 1)
    return pl.pallas_call(
        _layer_body,
        out_shape=jax.ShapeDtypeStruct((n, hid), jnp.float32),
    )(h, agg,
      eps2d, p['W1'], p['b1'].reshape(1, -1), p['bn_g'].reshape(1, -1),
      p['bn_b'].reshape(1, -1), p['W2'], p['b2'].reshape(1, -1),
      p['obn_g'].reshape(1, -1), p['obn_b'].reshape(1, -1))


def _head_body(h_ref, batch_ref, w1_ref, b1_ref, g1_ref, be1_ref,
               w2_ref, b2_ref, out_ref):
    gids = lax.broadcasted_iota(jnp.int32, (NUM_GRAPHS, NUM_NODES), 0)
    onehot = (batch_ref[...] == gids).astype(jnp.float32)
    g = jnp.dot(onehot, h_ref[...], preferred_element_type=jnp.float32)
    g = jnp.dot(g, w1_ref[...], preferred_element_type=jnp.float32) + b1_ref[...]
    g = _bn_cols(g, g1_ref[...], be1_ref[...])
    g = jnp.maximum(g, 0.0)
    g = jnp.dot(g, w2_ref[...], preferred_element_type=jnp.float32) + b2_ref[...]
    m = jnp.max(g, axis=1, keepdims=True)
    e = g - m
    lse = jnp.log(jnp.sum(jnp.exp(e), axis=1, keepdims=True))
    out_ref[...] = e - lse


def _head(h, batch, params):
    return pl.pallas_call(
        _head_body,
        out_shape=jax.ShapeDtypeStruct((NUM_GRAPHS, params['lin2_W'].shape[1]),
                                       jnp.float32),
    )(h, batch.reshape(1, NUM_NODES),
      params['lin1_W'], params['lin1_b'].reshape(1, -1),
      params['bn1_g'].reshape(1, -1), params['bn1_b'].reshape(1, -1),
      params['lin2_W'], params['lin2_b'].reshape(1, -1))


def kernel(x, edge_index, batch, params):
    src = edge_index[0]
    dst = edge_index[1]
    h = x
    for i in range(3):
        p = params['conv%d' % i]
        agg = jax.ops.segment_sum(h[src], dst, num_segments=NUM_NODES)
        h = _gin_layer(h, agg, p)
    return _head(h, batch, params)


# speedup vs baseline: 7.8893x; 7.8893x over previous
import functools

import jax
import jax.numpy as jnp
from jax import lax
from jax.experimental import pallas as pl
from jax.experimental.pallas import tpu as pltpu
from jax.experimental.pallas import tpu_sc as plsc

NUM_NODES = 10000
NUM_EDGES = 320000
NUM_GRAPHS = 64
HID = 256

NSUB = 16
NCORE = 2
BLK = 8
STRIPE = 624
TAIL_BASE = NSUB * STRIPE
TAIL = NUM_NODES - TAIL_BASE

CS_CHUNK = 100
CS_NCHUNK = NUM_EDGES // NSUB // CS_CHUNK
CS_NBLK = CS_NCHUNK // BLK
ES_CHUNK = 125
ES_NCHUNK = NUM_EDGES // (NCORE * NSUB) // ES_CHUNK
ES_NBLK = ES_NCHUNK // BLK


def _segsum_body(chan_split, chunk, nblk, h_hbm, src_hbm, dst_hbm, zeros_hbm,
                 out_hbm, src_set, dst_set, rows0, rows1, accum,
                 sem0, sem1, isem):
    c = lax.axis_index("c")
    s = lax.axis_index("s")
    if chan_split:
        h_c = h_hbm.at[c]
        widx = s
    else:
        h_c = h_hbm
        widx = c * NSUB + s

    pltpu.sync_copy(zeros_hbm.at[pl.ds(s * STRIPE, STRIPE)],
                    accum.at[pl.ds(s * STRIPE, STRIPE)])

    @pl.when(s == NSUB - 1)
    def _():
        pltpu.sync_copy(zeros_hbm.at[pl.ds(TAIL_BASE, TAIL)],
                        accum.at[pl.ds(TAIL_BASE, TAIL)])

    def idx_load(b, p):
        ds = pltpu.make_async_copy(src_hbm.at[widx, pl.ds(b * BLK, BLK)],
                                   src_set.at[p], isem.at[p])
        dd = pltpu.make_async_copy(dst_hbm.at[widx, pl.ds(b * BLK, BLK)],
                                   dst_set.at[p], isem.at[p])
        return ds, dd

    def gather(idx_row, buf, sem):
        return pltpu.make_async_copy(h_c.at[idx_row], buf, sem)

    ds0, dd0 = idx_load(0, 0)
    ds0.start()
    dd0.start()
    plsc.subcore_barrier()

    def block_step(b, carry):
        p = b % 2

        @pl.when(b + 1 < nblk)
        def _():
            ds, dd = idx_load(b + 1, 1 - p)
            ds.start()
            dd.start()

        dsw, ddw = idx_load(b, p)
        dsw.wait()
        ddw.wait()

        gather(src_set.at[p, 0], rows0, sem0).start()
        for k in range(BLK):
            buf, sem = (rows0, sem0) if k % 2 == 0 else (rows1, sem1)
            nbuf, nsem = (rows1, sem1) if k % 2 == 0 else (rows0, sem0)
            if k + 1 < BLK:
                gather(src_set.at[p, k + 1], nbuf, nsem).start()
            gather(src_set.at[p, k], buf, sem).wait()
            pltpu.sync_copy(buf, accum.at[dst_set.at[p, k]], add=True)
        return carry

    lax.fori_loop(0, nblk, block_step, 0)
    plsc.subcore_barrier()
    pltpu.sync_copy(accum.at[pl.ds(s * STRIPE, STRIPE)],
                    out_hbm.at[c].at[pl.ds(s * STRIPE, STRIPE)])

    @pl.when(s == NSUB - 1)
    def _():
        pltpu.sync_copy(accum.at[pl.ds(TAIL_BASE, TAIL)],
                        out_hbm.at[c].at[pl.ds(TAIL_BASE, TAIL)])


def _segsum_sc(chan_split, h_arr, src_r, dst_r, zeros):
    chunk = CS_CHUNK if chan_split else ES_CHUNK
    nblk = CS_NBLK if chan_split else ES_NBLK
    mesh = plsc.VectorSubcoreMesh(core_axis_name="c", subcore_axis_name="s")
    return pl.kernel(
        functools.partial(_segsum_body, chan_split, chunk, nblk),
        out_type=jax.ShapeDtypeStruct((NCORE, NUM_NODES, 128), jnp.float32),
        mesh=mesh,
        scratch_types=[
            pltpu.VMEM((2, BLK, chunk), jnp.int32),
            pltpu.VMEM((2, BLK, chunk), jnp.int32),
            pltpu.VMEM((chunk, 128), jnp.float32),
            pltpu.VMEM((chunk, 128), jnp.float32),
            pltpu.VMEM_SHARED((NUM_NODES, 128), jnp.float32),
            pltpu.SemaphoreType.DMA,
            pltpu.SemaphoreType.DMA,
            pltpu.SemaphoreType.DMA((2,)),
        ],
        name="segsum_sc",
    )(h_arr, src_r, dst_r, zeros)


def _bn_cols(t, gamma, beta, eps=1e-5):
    mean = jnp.mean(t, axis=0, keepdims=True)
    var = jnp.mean((t - mean) ** 2, axis=0, keepdims=True)
    return gamma * (t - mean) * lax.rsqrt(var + eps) + beta


def _layer_body(first, h_ref, agg_ref, eps_ref, w1_ref, b1_ref, g1_ref,
                be1_ref, w2_ref, b2_ref, g2_ref, be2_ref, out_ref):
    eps = eps_ref[0, 0]
    if first:
        h = h_ref[...]
        agg = agg_ref[0] + agg_ref[1]
    else:
        h = jnp.concatenate([h_ref[0], h_ref[1]], axis=1)
        agg = jnp.concatenate([agg_ref[0], agg_ref[1]], axis=1)
    z = (1.0 + eps) * h + agg
    t = jnp.dot(z, w1_ref[...], preferred_element_type=jnp.float32) + b1_ref[...]
    t = _bn_cols(t, g1_ref[...], be1_ref[...])
    t = jnp.maximum(t, 0.0)
    u = jnp.dot(t, w2_ref[...], preferred_element_type=jnp.float32) + b2_ref[...]
    u = _bn_cols(u, g2_ref[...], be2_ref[...])
    u = jnp.maximum(u, 0.0)
    out_ref[0] = u[:, :HID // 2]
    out_ref[1] = u[:, HID // 2:]


def _gin_layer(first, h_arr, agg_arr, p):
    eps2d = p['eps'].reshape(1, 1)
    return pl.pallas_call(
        functools.partial(_layer_body, first),
        out_shape=jax.ShapeDtypeStruct((2, NUM_NODES, HID // 2), jnp.float32),
    )(h_arr, agg_arr,
      eps2d, p['W1'], p['b1'].reshape(1, -1), p['bn_g'].reshape(1, -1),
      p['bn_b'].reshape(1, -1), p['W2'], p['b2'].reshape(1, -1),
      p['obn_g'].reshape(1, -1), p['obn_b'].reshape(1, -1))


def _head_body(h_ref, batch_ref, w1_ref, b1_ref, g1_ref, be1_ref,
               w2_ref, b2_ref, out_ref):
    h = jnp.concatenate([h_ref[0], h_ref[1]], axis=1)
    gids = lax.broadcasted_iota(jnp.int32, (NUM_GRAPHS, NUM_NODES), 0)
    onehot = (batch_ref[...] == gids).astype(jnp.float32)
    g = jnp.dot(onehot, h, preferred_element_type=jnp.float32)
    g = jnp.dot(g, w1_ref[...], preferred_element_type=jnp.float32) + b1_ref[...]
    g = _bn_cols(g, g1_ref[...], be1_ref[...])
    g = jnp.maximum(g, 0.0)
    g = jnp.dot(g, w2_ref[...], preferred_element_type=jnp.float32) + b2_ref[...]
    m = jnp.max(g, axis=1, keepdims=True)
    e = g - m
    lse = jnp.log(jnp.sum(jnp.exp(e), axis=1, keepdims=True))
    out_ref[...] = e - lse


def _head(h_split, batch, params):
    return pl.pallas_call(
        _head_body,
        out_shape=jax.ShapeDtypeStruct((NUM_GRAPHS, params['lin2_W'].shape[1]),
                                       jnp.float32),
    )(h_split, batch.reshape(1, NUM_NODES),
      params['lin1_W'], params['lin1_b'].reshape(1, -1),
      params['bn1_g'].reshape(1, -1), params['bn1_b'].reshape(1, -1),
      params['lin2_W'], params['lin2_b'].reshape(1, -1))


def kernel(x, edge_index, batch, params):
    src_es = edge_index[0].reshape(NCORE * NSUB, ES_NCHUNK, ES_CHUNK)
    dst_es = edge_index[1].reshape(NCORE * NSUB, ES_NCHUNK, ES_CHUNK)
    src_cs = edge_index[0].reshape(NSUB, CS_NCHUNK, CS_CHUNK)
    dst_cs = edge_index[1].reshape(NSUB, CS_NCHUNK, CS_CHUNK)
    zeros = jnp.zeros((NUM_NODES, 128), jnp.float32)

    agg2 = _segsum_sc(False, x, src_es, dst_es, zeros)
    h_split = _gin_layer(True, x, agg2, params['conv0'])

    for i in (1, 2):
        agg_split = _segsum_sc(True, h_split, src_cs, dst_cs, zeros)
        h_split = _gin_layer(False, h_split, agg_split, params['conv%d' % i])

    return _head(h_split, batch, params)


# speedup vs baseline: 8.1422x; 1.0321x over previous
import functools

import jax
import jax.numpy as jnp
from jax import lax
from jax.experimental import pallas as pl
from jax.experimental.pallas import tpu as pltpu
from jax.experimental.pallas import tpu_sc as plsc

NUM_NODES = 10000
NUM_EDGES = 320000
NUM_GRAPHS = 64
HID = 256

NSUB = 16
NCORE = 2
BLK = 8
STRIPE = 624
TAIL_BASE = NSUB * STRIPE
TAIL = NUM_NODES - TAIL_BASE

CS_CHUNK = 125
CS_NCHUNK = NUM_EDGES // NSUB // CS_CHUNK
CS_NBLK = CS_NCHUNK // BLK
ES_CHUNK = 125
ES_NCHUNK = NUM_EDGES // (NCORE * NSUB) // ES_CHUNK
ES_NBLK = ES_NCHUNK // BLK


def _segsum_body(chan_split, chunk, nblk, h_hbm, src_hbm, dst_hbm, zeros_hbm,
                 out_hbm, src_set, dst_set, rows0, rows1, accum,
                 sem0, sem1, isem):
    c = lax.axis_index("c")
    s = lax.axis_index("s")
    if chan_split:
        h_c = h_hbm.at[c]
        widx = s
    else:
        h_c = h_hbm
        widx = c * NSUB + s

    pltpu.sync_copy(zeros_hbm.at[pl.ds(s * STRIPE, STRIPE)],
                    accum.at[pl.ds(s * STRIPE, STRIPE)])

    @pl.when(s == NSUB - 1)
    def _():
        pltpu.sync_copy(zeros_hbm.at[pl.ds(TAIL_BASE, TAIL)],
                        accum.at[pl.ds(TAIL_BASE, TAIL)])

    def idx_load(b, p):
        ds = pltpu.make_async_copy(src_hbm.at[widx, pl.ds(b * BLK, BLK)],
                                   src_set.at[p], isem.at[p])
        dd = pltpu.make_async_copy(dst_hbm.at[widx, pl.ds(b * BLK, BLK)],
                                   dst_set.at[p], isem.at[p])
        return ds, dd

    def gather(idx_row, buf, sem):
        return pltpu.make_async_copy(h_c.at[idx_row], buf, sem)

    ds0, dd0 = idx_load(0, 0)
    ds0.start()
    dd0.start()
    plsc.subcore_barrier()

    def block_step(b, carry):
        p = b % 2

        @pl.when(b + 1 < nblk)
        def _():
            ds, dd = idx_load(b + 1, 1 - p)
            ds.start()
            dd.start()

        dsw, ddw = idx_load(b, p)
        dsw.wait()
        ddw.wait()

        gather(src_set.at[p, 0], rows0, sem0).start()
        for k in range(BLK):
            buf, sem = (rows0, sem0) if k % 2 == 0 else (rows1, sem1)
            nbuf, nsem = (rows1, sem1) if k % 2 == 0 else (rows0, sem0)
            if k + 1 < BLK:
                gather(src_set.at[p, k + 1], nbuf, nsem).start()
            gather(src_set.at[p, k], buf, sem).wait()
            pltpu.sync_copy(buf, accum.at[dst_set.at[p, k]], add=True)
        return carry

    lax.fori_loop(0, nblk, block_step, 0)
    plsc.subcore_barrier()
    pltpu.sync_copy(accum.at[pl.ds(s * STRIPE, STRIPE)],
                    out_hbm.at[c].at[pl.ds(s * STRIPE, STRIPE)])

    @pl.when(s == NSUB - 1)
    def _():
        pltpu.sync_copy(accum.at[pl.ds(TAIL_BASE, TAIL)],
                        out_hbm.at[c].at[pl.ds(TAIL_BASE, TAIL)])


def _segsum_sc(chan_split, h_arr, src_r, dst_r, zeros):
    chunk = CS_CHUNK if chan_split else ES_CHUNK
    nblk = CS_NBLK if chan_split else ES_NBLK
    mesh = plsc.VectorSubcoreMesh(core_axis_name="c", subcore_axis_name="s")
    return pl.kernel(
        functools.partial(_segsum_body, chan_split, chunk, nblk),
        out_type=jax.ShapeDtypeStruct((NCORE, NUM_NODES, 128), jnp.float32),
        mesh=mesh,
        scratch_types=[
            pltpu.VMEM((2, BLK, chunk), jnp.int32),
            pltpu.VMEM((2, BLK, chunk), jnp.int32),
            pltpu.VMEM((chunk, 128), jnp.float32),
            pltpu.VMEM((chunk, 128), jnp.float32),
            pltpu.VMEM_SHARED((NUM_NODES, 128), jnp.float32),
            pltpu.SemaphoreType.DMA,
            pltpu.SemaphoreType.DMA,
            pltpu.SemaphoreType.DMA((2,)),
        ],
        name="segsum_sc",
    )(h_arr, src_r, dst_r, zeros)


def _bn_cols(t, gamma, beta, eps=1e-5):
    mean = jnp.mean(t, axis=0, keepdims=True)
    var = jnp.mean((t - mean) ** 2, axis=0, keepdims=True)
    return gamma * (t - mean) * lax.rsqrt(var + eps) + beta


def _layer_body(first, h_ref, agg_ref, eps_ref, w1_ref, b1_ref, g1_ref,
                be1_ref, w2_ref, b2_ref, g2_ref, be2_ref, out_ref):
    eps = eps_ref[0, 0]
    if first:
        h = h_ref[...]
        agg = agg_ref[0] + agg_ref[1]
    else:
        h = jnp.concatenate([h_ref[0], h_ref[1]], axis=1)
        agg = jnp.concatenate([agg_ref[0], agg_ref[1]], axis=1)
    z = (1.0 + eps) * h + agg
    t = jnp.dot(z, w1_ref[...], preferred_element_type=jnp.float32) + b1_ref[...]
    t = _bn_cols(t, g1_ref[...], be1_ref[...])
    t = jnp.maximum(t, 0.0)
    u = jnp.dot(t, w2_ref[...], preferred_element_type=jnp.float32) + b2_ref[...]
    u = _bn_cols(u, g2_ref[...], be2_ref[...])
    u = jnp.maximum(u, 0.0)
    out_ref[0] = u[:, :HID // 2]
    out_ref[1] = u[:, HID // 2:]


def _gin_layer(first, h_arr, agg_arr, p):
    eps2d = p['eps'].reshape(1, 1)
    return pl.pallas_call(
        functools.partial(_layer_body, first),
        out_shape=jax.ShapeDtypeStruct((2, NUM_NODES, HID // 2), jnp.float32),
    )(h_arr, agg_arr,
      eps2d, p['W1'], p['b1'].reshape(1, -1), p['bn_g'].reshape(1, -1),
      p['bn_b'].reshape(1, -1), p['W2'], p['b2'].reshape(1, -1),
      p['obn_g'].reshape(1, -1), p['obn_b'].reshape(1, -1))


def _head_body(h_ref, batch_ref, w1_ref, b1_ref, g1_ref, be1_ref,
               w2_ref, b2_ref, out_ref):
    h = jnp.concatenate([h_ref[0], h_ref[1]], axis=1)
    gids = lax.broadcasted_iota(jnp.int32, (NUM_GRAPHS, NUM_NODES), 0)
    onehot = (batch_ref[...] == gids).astype(jnp.float32)
    g = jnp.dot(onehot, h, preferred_element_type=jnp.float32)
    g = jnp.dot(g, w1_ref[...], preferred_element_type=jnp.float32) + b1_ref[...]
    g = _bn_cols(g, g1_ref[...], be1_ref[...])
    g = jnp.maximum(g, 0.0)
    g = jnp.dot(g, w2_ref[...], preferred_element_type=jnp.float32) + b2_ref[...]
    m = jnp.max(g, axis=1, keepdims=True)
    e = g - m
    lse = jnp.log(jnp.sum(jnp.exp(e), axis=1, keepdims=True))
    out_ref[...] = e - lse


def _head(h_split, batch, params):
    return pl.pallas_call(
        _head_body,
        out_shape=jax.ShapeDtypeStruct((NUM_GRAPHS, params['lin2_W'].shape[1]),
                                       jnp.float32),
    )(h_split, batch.reshape(1, NUM_NODES),
      params['lin1_W'], params['lin1_b'].reshape(1, -1),
      params['bn1_g'].reshape(1, -1), params['bn1_b'].reshape(1, -1),
      params['lin2_W'], params['lin2_b'].reshape(1, -1))


def kernel(x, edge_index, batch, params):
    src_es = edge_index[0].reshape(NCORE * NSUB, ES_NCHUNK, ES_CHUNK)
    dst_es = edge_index[1].reshape(NCORE * NSUB, ES_NCHUNK, ES_CHUNK)
    src_cs = edge_index[0].reshape(NSUB, CS_NCHUNK, CS_CHUNK)
    dst_cs = edge_index[1].reshape(NSUB, CS_NCHUNK, CS_CHUNK)
    zeros = jnp.zeros((NUM_NODES, 128), jnp.float32)

    agg2 = _segsum_sc(False, x, src_es, dst_es, zeros)
    h_split = _gin_layer(True, x, agg2, params['conv0'])

    for i in (1, 2):
        agg_split = _segsum_sc(True, h_split, src_cs, dst_cs, zeros)
        h_split = _gin_layer(False, h_split, agg_split, params['conv%d' % i])

    return _head(h_split, batch, params)
